# baseline (speedup 1.0000x reference)
import functools

import jax
import jax.numpy as jnp
from jax.experimental import pallas as pl
from jax.experimental.pallas import tpu as pltpu

R, C = 32, 1_000_000
BW = 32_768
NBLK = (C + BW - 1) // BW


def _masked_max_body(label_ref, x_ref, o_ref):
    pid = pl.program_id(0)
    label = label_ref[0]
    x = x_ref[...]
    col = pid * BW + jax.lax.broadcasted_iota(jnp.int32, (R, BW), 1)
    valid = (col < C) & (x >= label)
    local = jnp.max(jnp.where(valid, x, -jnp.inf))

    @pl.when(pid == 0)
    def _init():
        o_ref[0, 0] = local

    @pl.when(pid > 0)
    def _acc():
        o_ref[0, 0] = jnp.maximum(o_ref[0, 0], local)


def kernel(logits, label_logit):
    masked_max = pl.pallas_call(
        _masked_max_body,
        grid=(NBLK,),
        in_specs=[
            pl.BlockSpec(memory_space=pltpu.SMEM),
            pl.BlockSpec((R, BW), lambda i: (0, i)),
        ],
        out_specs=pl.BlockSpec(memory_space=pltpu.SMEM),
        out_shape=jax.ShapeDtypeStruct((1, 1), jnp.float32),
    )(label_logit, logits)
    return label_logit - masked_max[0]
